# baseline (speedup 1.0000x reference)
import functools

import jax
import jax.numpy as jnp
from jax import lax
from jax.experimental import pallas as pl
from jax.experimental.pallas import tpu as pltpu
from jax.experimental.pallas import tpu_sc as plsc

B = 16384
D = 128
H = 20
UF = 64
CD = 128

NC = 2
NS = 16
NW = NC * NS
CHUNK = 16
TAG_GRP = 80
TAG_GRPS_PER_CHUNK = CHUNK * H // TAG_GRP

SLICES = (13312, 3072)


def _make_sc_body(bpw, nchunk):
    def body(ui_h, ti_h, ii_h, ut_h, tt_h, it_h,
             ou_h, ot_h, oi_h,
             idxu_v, idxi_v, idxt_v,
             rows_t0, rows_t1, rows_u0, rows_u1, rows_i0, rows_i1,
             acc_v0, acc_v1,
             sem_g0, sem_g1, sem_w0, sem_w1):
        wid = lax.axis_index("s") * NC + lax.axis_index("c")
        base_out = wid * bpw
        pltpu.sync_copy(ui_h.at[pl.ds(base_out, bpw)], idxu_v)
        pltpu.sync_copy(ii_h.at[pl.ds(base_out, bpw)], idxi_v)
        pltpu.sync_copy(ti_h.at[wid], idxt_v)

        rows_t = (rows_t0, rows_t1)
        rows_u = (rows_u0, rows_u1)
        rows_i = (rows_i0, rows_i1)
        acc_v = (acc_v0, acc_v1)
        sem_g = (sem_g0, sem_g1)
        sem_w = (sem_w0, sem_w1)

        def issue_gathers(c, p):
            for g in range(TAG_GRPS_PER_CHUNK):
                pltpu.async_copy(tt_h.at[idxt_v.at[c * TAG_GRPS_PER_CHUNK + g]],
                                 rows_t[p].at[pl.ds(g * TAG_GRP, TAG_GRP)],
                                 sem_g[p])
            pltpu.async_copy(ut_h.at[idxu_v.at[pl.ds(c * CHUNK, CHUNK)]],
                             rows_u[p], sem_g[p])
            pltpu.async_copy(it_h.at[idxi_v.at[pl.ds(c * CHUNK, CHUNK)]],
                             rows_i[p], sem_g[p])

        def wait_gathers(p):
            for g in range(TAG_GRPS_PER_CHUNK):
                pltpu.make_async_copy(tt_h.at[idxt_v.at[0]],
                                      rows_t[p].at[pl.ds(g * TAG_GRP, TAG_GRP)],
                                      sem_g[p]).wait()
            pltpu.make_async_copy(ut_h.at[idxu_v.at[pl.ds(0, CHUNK)]],
                                  rows_u[p], sem_g[p]).wait()
            pltpu.make_async_copy(it_h.at[idxi_v.at[pl.ds(0, CHUNK)]],
                                  rows_i[p], sem_g[p]).wait()

        def issue_writes(c, p):
            dst = pl.ds(base_out + c * CHUNK, CHUNK)
            pltpu.async_copy(rows_u[p], ou_h.at[dst], sem_w[p])
            pltpu.async_copy(rows_i[p], oi_h.at[dst], sem_w[p])
            pltpu.async_copy(acc_v[p], ot_h.at[dst], sem_w[p])

        def wait_writes(p):
            dst = pl.ds(base_out, CHUNK)
            pltpu.make_async_copy(rows_u[p], ou_h.at[dst], sem_w[p]).wait()
            pltpu.make_async_copy(rows_i[p], oi_h.at[dst], sem_w[p]).wait()
            pltpu.make_async_copy(acc_v[p], ot_h.at[dst], sem_w[p]).wait()

        def accumulate(p):
            rt = rows_t[p]
            av = acc_v[p]

            def samp(s, carry2):
                r0 = s * H
                for d in range(D // 16):
                    sl = pl.ds(d * 16, 16)
                    v = rt[r0, sl]
                    for h in range(1, H):
                        v = v + rt[r0 + h, sl]
                    av[s, sl] = v
                return carry2

            lax.fori_loop(0, CHUNK, samp, 0, unroll=False)

        issue_gathers(0, 0)

        def outer(t, carry):
            for p in (0, 1):
                c = 2 * t + p

                @pl.when(c >= 1)
                def _():
                    wait_writes(1 - p)

                @pl.when(c + 1 < nchunk)
                def _():
                    issue_gathers(c + 1, 1 - p)

                wait_gathers(p)
                accumulate(p)
                issue_writes(c, p)
            return carry

        lax.fori_loop(0, nchunk // 2, outer, 0, unroll=False)
        wait_writes(1)

    return body


@functools.cache
def _sc_gather_fn(bs):
    bpw = bs // NW
    nchunk = bpw // CHUNK
    ngrp = nchunk * TAG_GRPS_PER_CHUNK
    f32 = jnp.float32
    return pl.kernel(
        _make_sc_body(bpw, nchunk),
        out_type=(
            jax.ShapeDtypeStruct((bs, D), f32),
            jax.ShapeDtypeStruct((bs, D), f32),
            jax.ShapeDtypeStruct((bs, D), f32),
        ),
        mesh=plsc.VectorSubcoreMesh(core_axis_name="c", subcore_axis_name="s"),
        scratch_types=(
            pltpu.VMEM((bpw,), jnp.int32),
            pltpu.VMEM((bpw,), jnp.int32),
            pltpu.VMEM((ngrp, TAG_GRP), jnp.int32),
            pltpu.VMEM((CHUNK * H, D), f32),
            pltpu.VMEM((CHUNK * H, D), f32),
            pltpu.VMEM((CHUNK, D), f32),
            pltpu.VMEM((CHUNK, D), f32),
            pltpu.VMEM((CHUNK, D), f32),
            pltpu.VMEM((CHUNK, D), f32),
            pltpu.VMEM((CHUNK, D), f32),
            pltpu.VMEM((CHUNK, D), f32),
            pltpu.SemaphoreType.DMA,
            pltpu.SemaphoreType.DMA,
            pltpu.SemaphoreType.DMA,
            pltpu.SemaphoreType.DMA,
        ),
    )


def _dotT(x, w):
    return lax.dot_general(x, w, (((1,), (1,)), ((), ())),
                           preferred_element_type=jnp.float32)


def _layernorm(x, g, b):
    m = jnp.mean(x, axis=-1, keepdims=True)
    v = jnp.mean((x - m) ** 2, axis=-1, keepdims=True)
    return (x - m) * lax.rsqrt(v + 1e-5) * g + b


BT = 1024


def _tc_body(uid, tagsum, iid, cf,
             tpW, tpb, tpg, tpbeta,
             ufW, ufb, ufg, ufbeta,
             cfW, cfb, cfg, cfbeta,
             ifW, ifb, ifg, ifbeta,
             m1W, m1b, m2W, m2b, m3W, m3b, pW, pb,
             out):
    tag = tagsum[...] * (1.0 / H)
    t1 = _layernorm(jax.nn.relu(_dotT(tag, tpW[...]) + tpb[...]),
                    tpg[...], tpbeta[...])
    ufw = ufW[...]
    ue = _layernorm(
        jax.nn.relu(_dotT(uid[...], ufw[:, :D]) + _dotT(t1, ufw[:, D:]) + ufb[...]),
        ufg[...], ufbeta[...])
    ce = _layernorm(jax.nn.relu(_dotT(cf[...], cfW[...]) + cfb[...]),
                    cfg[...], cfbeta[...])
    ifw = ifW[...]
    ie = _layernorm(
        jax.nn.relu(_dotT(iid[...], ifw[:, :D]) + _dotT(ce, ifw[:, D:]) + ifb[...]),
        ifg[...], ifbeta[...])
    m1w = m1W[...]
    h = jax.nn.relu(_dotT(ue, m1w[:, :UF]) + _dotT(ie, m1w[:, UF:]) + m1b[...])
    h = jax.nn.relu(_dotT(h, m2W[...]) + m2b[...])
    h = jax.nn.relu(_dotT(h, m3W[...]) + m3b[...])
    logit = jnp.sum(h * pW[...], axis=1, keepdims=True) + pb[0, 0]
    out[...] = 1.0 / (1.0 + jnp.exp(-logit))


def _tc_dense(uid, tagsum, iid, cf, *weights):
    f32 = jnp.float32
    bs = uid.shape[0]
    grid = bs // BT
    row_spec = pl.BlockSpec((BT, D), lambda i: (i, 0))

    def w_spec(w):
        return pl.BlockSpec(w.shape, lambda i: tuple(0 for _ in w.shape))

    return pl.pallas_call(
        _tc_body,
        grid=(grid,),
        in_specs=[row_spec, row_spec, row_spec, row_spec] +
                 [w_spec(w) for w in weights],
        out_specs=pl.BlockSpec((BT, 1), lambda i: (i, 0)),
        out_shape=jax.ShapeDtypeStruct((bs, 1), f32),
        compiler_params=pltpu.CompilerParams(
            dimension_semantics=("arbitrary",),
        ),
    )(uid, tagsum, iid, cf, *weights)


def kernel(user_ids, user_tags_idx, item_ids, content_features, user_table,
           tag_table, item_table, tp_W, tp_b, tp_g, tp_beta, uf_W, uf_b, uf_g,
           uf_beta, cf_W, cf_b, cf_g, cf_beta, if_W, if_b, if_g, if_beta,
           m1_W, m1_b, m2_W, m2_b, m3_W, m3_b, p_W, p_b):
    ui = user_ids.astype(jnp.int32)
    ii = item_ids.astype(jnp.int32)
    ti = user_tags_idx.astype(jnp.int32)

    def r2(v):
        return v.reshape(1, -1)

    weights = (tp_W, r2(tp_b), r2(tp_g), r2(tp_beta),
               uf_W, r2(uf_b), r2(uf_g), r2(uf_beta),
               cf_W, r2(cf_b), r2(cf_g), r2(cf_beta),
               if_W, r2(if_b), r2(if_g), r2(if_beta),
               m1_W, r2(m1_b), m2_W, r2(m2_b), m3_W, r2(m3_b), p_W, r2(p_b))

    outs = []
    off = 0
    for bs in SLICES:
        sl = slice(off, off + bs)
        off += bs
        bpw = bs // NW
        ngrp = (bpw // CHUNK) * TAG_GRPS_PER_CHUNK
        ti3 = ti[sl].reshape(NW, ngrp, TAG_GRP)
        uid, tagsum, iid = _sc_gather_fn(bs)(ui[sl], ti3, ii[sl], user_table,
                                             tag_table, item_table)
        outs.append(_tc_dense(uid, tagsum, iid, content_features[sl],
                              *weights))
    return jnp.concatenate(outs, axis=0)[:, 0]


# speedup vs baseline: 1.0306x; 1.0306x over previous
import functools

import jax
import jax.numpy as jnp
from jax import lax
from jax.experimental import pallas as pl
from jax.experimental.pallas import tpu as pltpu
from jax.experimental.pallas import tpu_sc as plsc

B = 16384
D = 128
H = 20
UF = 64
CD = 128

NC = 2
NS = 16
NW = NC * NS
CHUNK = 16
TAG_GRP = 80
TAG_GRPS_PER_CHUNK = CHUNK * H // TAG_GRP

SLICES = (12288, 4096)


def _make_sc_body(bpw, nchunk):
    def body(ui_h, ti_h, ii_h, ut_h, tt_h, it_h,
             ou_h, ot_h, oi_h,
             idxu_v, idxi_v, idxt_v,
             rows_t0, rows_t1, rows_u0, rows_u1, rows_i0, rows_i1,
             acc_v0, acc_v1,
             sem_g0, sem_g1, sem_w0, sem_w1):
        wid = lax.axis_index("s") * NC + lax.axis_index("c")
        base_out = wid * bpw
        pltpu.async_copy(ui_h.at[pl.ds(base_out, bpw)], idxu_v, sem_w0)
        pltpu.async_copy(ii_h.at[pl.ds(base_out, bpw)], idxi_v, sem_w0)
        pltpu.async_copy(ti_h.at[wid], idxt_v, sem_w0)
        pltpu.make_async_copy(ui_h.at[pl.ds(base_out, bpw)], idxu_v,
                              sem_w0).wait()
        pltpu.make_async_copy(ii_h.at[pl.ds(base_out, bpw)], idxi_v,
                              sem_w0).wait()
        pltpu.make_async_copy(ti_h.at[wid], idxt_v, sem_w0).wait()

        rows_t = (rows_t0, rows_t1)
        rows_u = (rows_u0, rows_u1)
        rows_i = (rows_i0, rows_i1)
        acc_v = (acc_v0, acc_v1)
        sem_g = (sem_g0, sem_g1)
        sem_w = (sem_w0, sem_w1)

        def issue_gathers(c, p):
            for g in range(TAG_GRPS_PER_CHUNK):
                pltpu.async_copy(tt_h.at[idxt_v.at[c * TAG_GRPS_PER_CHUNK + g]],
                                 rows_t[p].at[pl.ds(g * TAG_GRP, TAG_GRP)],
                                 sem_g[p])
            pltpu.async_copy(ut_h.at[idxu_v.at[pl.ds(c * CHUNK, CHUNK)]],
                             rows_u[p], sem_g[p])
            pltpu.async_copy(it_h.at[idxi_v.at[pl.ds(c * CHUNK, CHUNK)]],
                             rows_i[p], sem_g[p])

        def wait_gathers(p):
            for g in range(TAG_GRPS_PER_CHUNK):
                pltpu.make_async_copy(tt_h.at[idxt_v.at[0]],
                                      rows_t[p].at[pl.ds(g * TAG_GRP, TAG_GRP)],
                                      sem_g[p]).wait()
            pltpu.make_async_copy(ut_h.at[idxu_v.at[pl.ds(0, CHUNK)]],
                                  rows_u[p], sem_g[p]).wait()
            pltpu.make_async_copy(it_h.at[idxi_v.at[pl.ds(0, CHUNK)]],
                                  rows_i[p], sem_g[p]).wait()

        def issue_writes(c, p):
            dst = pl.ds(base_out + c * CHUNK, CHUNK)
            pltpu.async_copy(rows_u[p], ou_h.at[dst], sem_w[p])
            pltpu.async_copy(rows_i[p], oi_h.at[dst], sem_w[p])
            pltpu.async_copy(acc_v[p], ot_h.at[dst], sem_w[p])

        def wait_writes(p):
            dst = pl.ds(base_out, CHUNK)
            pltpu.make_async_copy(rows_u[p], ou_h.at[dst], sem_w[p]).wait()
            pltpu.make_async_copy(rows_i[p], oi_h.at[dst], sem_w[p]).wait()
            pltpu.make_async_copy(acc_v[p], ot_h.at[dst], sem_w[p]).wait()

        def accumulate(p):
            rt = rows_t[p]
            av = acc_v[p]

            def samp(s, carry2):
                r0 = s * H
                for d in range(D // 16):
                    sl = pl.ds(d * 16, 16)
                    v = rt[r0, sl]
                    for h in range(1, H):
                        v = v + rt[r0 + h, sl]
                    av[s, sl] = v
                return carry2

            lax.fori_loop(0, CHUNK, samp, 0, unroll=False)

        issue_gathers(0, 0)

        def outer(t, carry):
            for p in (0, 1):
                c = 2 * t + p

                @pl.when(c >= 1)
                def _():
                    wait_writes(1 - p)

                @pl.when(c + 1 < nchunk)
                def _():
                    issue_gathers(c + 1, 1 - p)

                wait_gathers(p)
                accumulate(p)
                issue_writes(c, p)
            return carry

        lax.fori_loop(0, nchunk // 2, outer, 0, unroll=False)
        wait_writes(1)

    return body


@functools.cache
def _sc_gather_fn(bs):
    bpw = bs // NW
    nchunk = bpw // CHUNK
    ngrp = nchunk * TAG_GRPS_PER_CHUNK
    f32 = jnp.float32
    return pl.kernel(
        _make_sc_body(bpw, nchunk),
        out_type=(
            jax.ShapeDtypeStruct((bs, D), f32),
            jax.ShapeDtypeStruct((bs, D), f32),
            jax.ShapeDtypeStruct((bs, D), f32),
        ),
        mesh=plsc.VectorSubcoreMesh(core_axis_name="c", subcore_axis_name="s"),
        scratch_types=(
            pltpu.VMEM((bpw,), jnp.int32),
            pltpu.VMEM((bpw,), jnp.int32),
            pltpu.VMEM((ngrp, TAG_GRP), jnp.int32),
            pltpu.VMEM((CHUNK * H, D), f32),
            pltpu.VMEM((CHUNK * H, D), f32),
            pltpu.VMEM((CHUNK, D), f32),
            pltpu.VMEM((CHUNK, D), f32),
            pltpu.VMEM((CHUNK, D), f32),
            pltpu.VMEM((CHUNK, D), f32),
            pltpu.VMEM((CHUNK, D), f32),
            pltpu.VMEM((CHUNK, D), f32),
            pltpu.SemaphoreType.DMA,
            pltpu.SemaphoreType.DMA,
            pltpu.SemaphoreType.DMA,
            pltpu.SemaphoreType.DMA,
        ),
    )


def _dotT(x, w):
    return lax.dot_general(x, w, (((1,), (1,)), ((), ())),
                           preferred_element_type=jnp.float32)


def _layernorm(x, g, b):
    m = jnp.mean(x, axis=-1, keepdims=True)
    v = jnp.mean((x - m) ** 2, axis=-1, keepdims=True)
    return (x - m) * lax.rsqrt(v + 1e-5) * g + b


BT = 1024


def _tc_body(uid, tagsum, iid, cf,
             tpW, tpb, tpg, tpbeta,
             ufW, ufb, ufg, ufbeta,
             cfW, cfb, cfg, cfbeta,
             ifW, ifb, ifg, ifbeta,
             m1W, m1b, m2W, m2b, m3W, m3b, pW, pb,
             out):
    tag = tagsum[...] * (1.0 / H)
    t1 = _layernorm(jax.nn.relu(_dotT(tag, tpW[...]) + tpb[...]),
                    tpg[...], tpbeta[...])
    ufw = ufW[...]
    ue = _layernorm(
        jax.nn.relu(_dotT(uid[...], ufw[:, :D]) + _dotT(t1, ufw[:, D:]) + ufb[...]),
        ufg[...], ufbeta[...])
    ce = _layernorm(jax.nn.relu(_dotT(cf[...], cfW[...]) + cfb[...]),
                    cfg[...], cfbeta[...])
    ifw = ifW[...]
    ie = _layernorm(
        jax.nn.relu(_dotT(iid[...], ifw[:, :D]) + _dotT(ce, ifw[:, D:]) + ifb[...]),
        ifg[...], ifbeta[...])
    m1w = m1W[...]
    h = jax.nn.relu(_dotT(ue, m1w[:, :UF]) + _dotT(ie, m1w[:, UF:]) + m1b[...])
    h = jax.nn.relu(_dotT(h, m2W[...]) + m2b[...])
    h = jax.nn.relu(_dotT(h, m3W[...]) + m3b[...])
    logit = jnp.sum(h * pW[...], axis=1, keepdims=True) + pb[0, 0]
    out[...] = 1.0 / (1.0 + jnp.exp(-logit))


def _tc_dense(uid, tagsum, iid, cf, *weights):
    f32 = jnp.float32
    bs = uid.shape[0]
    grid = bs // BT
    row_spec = pl.BlockSpec((BT, D), lambda i: (i, 0))

    def w_spec(w):
        return pl.BlockSpec(w.shape, lambda i: tuple(0 for _ in w.shape))

    return pl.pallas_call(
        _tc_body,
        grid=(grid,),
        in_specs=[row_spec, row_spec, row_spec, row_spec] +
                 [w_spec(w) for w in weights],
        out_specs=pl.BlockSpec((BT, 1), lambda i: (i, 0)),
        out_shape=jax.ShapeDtypeStruct((bs, 1), f32),
        compiler_params=pltpu.CompilerParams(
            dimension_semantics=("arbitrary",),
        ),
    )(uid, tagsum, iid, cf, *weights)


def kernel(user_ids, user_tags_idx, item_ids, content_features, user_table,
           tag_table, item_table, tp_W, tp_b, tp_g, tp_beta, uf_W, uf_b, uf_g,
           uf_beta, cf_W, cf_b, cf_g, cf_beta, if_W, if_b, if_g, if_beta,
           m1_W, m1_b, m2_W, m2_b, m3_W, m3_b, p_W, p_b):
    ui = user_ids.astype(jnp.int32)
    ii = item_ids.astype(jnp.int32)
    ti = user_tags_idx.astype(jnp.int32)

    def r2(v):
        return v.reshape(1, -1)

    weights = (tp_W, r2(tp_b), r2(tp_g), r2(tp_beta),
               uf_W, r2(uf_b), r2(uf_g), r2(uf_beta),
               cf_W, r2(cf_b), r2(cf_g), r2(cf_beta),
               if_W, r2(if_b), r2(if_g), r2(if_beta),
               m1_W, r2(m1_b), m2_W, r2(m2_b), m3_W, r2(m3_b), p_W, r2(p_b))

    outs = []
    off = 0
    for bs in SLICES:
        sl = slice(off, off + bs)
        off += bs
        bpw = bs // NW
        ngrp = (bpw // CHUNK) * TAG_GRPS_PER_CHUNK
        ti3 = ti[sl].reshape(NW, ngrp, TAG_GRP)
        uid, tagsum, iid = _sc_gather_fn(bs)(ui[sl], ti3, ii[sl], user_table,
                                             tag_table, item_table)
        outs.append(_tc_dense(uid, tagsum, iid, content_features[sl],
                              *weights))
    return jnp.concatenate(outs, axis=0)[:, 0]
